# Initial kernel scaffold; baseline (speedup 1.0000x reference)
import jax
import jax.numpy as jnp
from jax.experimental import pallas as pl


def kernel(h, edge_index, bases, params):
    raise NotImplementedError("write your pallas kernel here")


# speedup vs baseline: 2.8569x; 2.8569x over previous
import functools

import jax
import jax.numpy as jnp
from jax import lax
from jax.experimental import pallas as pl
from jax.experimental.pallas import tpu as pltpu
from jax.experimental.pallas import tpu_sc as plsc

_N = 10000
_E = 320000
_DIN = 128
_H = 256
_NB = 16
_HC = 128
_NCORE = 2
_NSUB = 16
_K = 80
_EPT = _E // _NSUB
_NPAD = 10240
_RPT = _NPAD // _NSUB
_BE = 8000


def _gelu(x):
    return 0.5 * x * (1.0 + lax.erf(x * 0.7071067811865476))


def _stats1_body(bases_ref, w1_ref, b1_ref, out_ref):
    i = pl.program_id(0)
    z = jnp.dot(bases_ref[...], w1_ref[...],
                preferred_element_type=jnp.float32) + b1_ref[...]

    @pl.when(i == 0)
    def _():
        out_ref[...] = jnp.zeros_like(out_ref)

    out_ref[0:1, :] += jnp.sum(z, axis=0, keepdims=True)
    out_ref[1:2, :] += jnp.sum(z * z, axis=0, keepdims=True)


def _stats2_body(bases_ref, w1_ref, b1_ref, a1_ref, c1_ref, w2_ref, b2_ref,
                 out_ref):
    i = pl.program_id(0)
    z1 = jnp.dot(bases_ref[...], w1_ref[...],
                 preferred_element_type=jnp.float32) + b1_ref[...]
    y1 = _gelu(z1 * a1_ref[...] + c1_ref[...])
    z2 = jnp.dot(y1, w2_ref[...],
                 preferred_element_type=jnp.float32) + b2_ref[...]

    @pl.when(i == 0)
    def _():
        out_ref[...] = jnp.zeros_like(out_ref)

    out_ref[0:1, :] += jnp.sum(z2, axis=0, keepdims=True)
    out_ref[1:2, :] += jnp.sum(z2 * z2, axis=0, keepdims=True)


def _edge_e_body(bases_ref, w1_ref, b1_ref, a1_ref, c1_ref, w2_ref, b2_ref,
                 a2_ref, c2_ref, e0_ref, e1_ref):
    z1 = jnp.dot(bases_ref[...], w1_ref[...],
                 preferred_element_type=jnp.float32) + b1_ref[...]
    y1 = _gelu(z1 * a1_ref[...] + c1_ref[...])
    z2 = jnp.dot(y1, w2_ref[...],
                 preferred_element_type=jnp.float32) + b2_ref[...]
    e = jnp.exp(_gelu(z2 * a2_ref[...] + c2_ref[...]))
    e0_ref[...] = e[:, :_HC]
    e1_ref[...] = e[:, _HC:]


def _node0_body(h_ref, w0_ref, b0_ref, wp_ref, bp_ref,
                h0_ref, xp0_ref, xp1_ref):
    h0 = jnp.dot(h_ref[...], w0_ref[...],
                 preferred_element_type=jnp.float32) + b0_ref[...]
    h0_ref[...] = h0
    xp = _gelu(jnp.dot(h0, wp_ref[...],
                       preferred_element_type=jnp.float32) + bp_ref[...])
    xp0_ref[...] = xp[:, :_HC]
    xp1_ref[...] = xp[:, _HC:]


def _ffn_a_body(h_ref, y0_ref, y1_ref, s0_ref, s1_ref, wf1_ref, bf1_ref,
                x_ref, z1_ref, st_ref):
    s0 = s0_ref[...]
    s1 = s1_ref[...]
    yn0 = y0_ref[...] * jnp.where(s0 > 0, 1.0 / s0, 0.0)
    yn1 = y1_ref[...] * jnp.where(s1 > 0, 1.0 / s1, 0.0)
    x = h_ref[...] + jnp.concatenate([yn0, yn1], axis=1)
    x_ref[...] = x
    z1 = jnp.dot(x, wf1_ref[...],
                 preferred_element_type=jnp.float32) + bf1_ref[...]
    z1_ref[...] = z1

    @pl.when(pl.program_id(0) == 0)
    def _():
        st_ref[...] = jnp.zeros_like(st_ref)

    st_ref[0:1, :] += jnp.sum(z1, axis=0, keepdims=True)
    st_ref[1:2, :] += jnp.sum(z1 * z1, axis=0, keepdims=True)


def _ffn_b_body(z1_ref, a_ref, c_ref, wf2_ref, bf2_ref, z2_ref, st_ref):
    t = _gelu(z1_ref[...] * a_ref[...] + c_ref[...])
    z2 = jnp.dot(t, wf2_ref[...],
                 preferred_element_type=jnp.float32) + bf2_ref[...]
    z2_ref[...] = z2

    @pl.when(pl.program_id(0) == 0)
    def _():
        st_ref[...] = jnp.zeros_like(st_ref)

    st_ref[0:1, :] += jnp.sum(z2, axis=0, keepdims=True)
    st_ref[1:2, :] += jnp.sum(z2 * z2, axis=0, keepdims=True)


def _ffn_c_body(x_ref, z2_ref, a_ref, c_ref, wp_ref, bp_ref,
                h_ref, xp0_ref, xp1_ref):
    hn = x_ref[...] + _gelu(z2_ref[...] * a_ref[...] + c_ref[...])
    h_ref[...] = hn
    xp = _gelu(jnp.dot(hn, wp_ref[...],
                       preferred_element_type=jnp.float32) + bp_ref[...])
    xp0_ref[...] = xp[:, :_HC]
    xp1_ref[...] = xp[:, _HC:]


def _ffn_cl_body(x_ref, z2_ref, a_ref, c_ref, nr_ref):
    hn = x_ref[...] + _gelu(z2_ref[...] * a_ref[...] + c_ref[...])

    @pl.when(pl.program_id(0) == 0)
    def _():
        nr_ref[...] = jnp.zeros_like(nr_ref)

    nr_ref[0:1, :] += jnp.sum(hn, axis=0, keepdims=True)


def _head_body(nr_ref, w1_ref, b1_ref, w2_ref, b2_ref, out_ref):
    nr = _gelu(jnp.dot(nr_ref[0:1, :], w1_ref[...],
                       preferred_element_type=jnp.float32) + b1_ref[...])
    out_ref[...] = jnp.dot(nr, w2_ref[...],
                           preferred_element_type=jnp.float32) + b2_ref[...]


def _vec_spec(n=_H):
    return pl.BlockSpec((1, n), lambda i: (0, 0))


def _full(shape):
    return pl.BlockSpec(shape, lambda i: tuple(0 for _ in shape))


_SEQ = pltpu.CompilerParams(dimension_semantics=("arbitrary",))


def _edge_stats1(bases, w1, b1):
    return pl.pallas_call(
        _stats1_body,
        grid=(_E // _BE,),
        in_specs=[pl.BlockSpec((_BE, _NB), lambda i: (i, 0)),
                  _full((_NB, _H)), _vec_spec()],
        out_specs=_full((8, _H)),
        out_shape=jax.ShapeDtypeStruct((8, _H), jnp.float32),
        compiler_params=_SEQ,
    )(bases, w1, b1)


def _edge_stats2(bases, w1, b1, a1, c1, w2, b2):
    return pl.pallas_call(
        _stats2_body,
        grid=(_E // _BE,),
        in_specs=[pl.BlockSpec((_BE, _NB), lambda i: (i, 0)),
                  _full((_NB, _H)), _vec_spec(), _vec_spec(), _vec_spec(),
                  _full((_H, _H)), _vec_spec()],
        out_specs=_full((8, _H)),
        out_shape=jax.ShapeDtypeStruct((8, _H), jnp.float32),
        compiler_params=_SEQ,
    )(bases, w1, b1, a1, c1, w2, b2)


def _edge_e(bases, w1, b1, a1, c1, w2, b2, a2, c2):
    return pl.pallas_call(
        _edge_e_body,
        grid=(_E // _BE,),
        in_specs=[pl.BlockSpec((_BE, _NB), lambda i: (i, 0)),
                  _full((_NB, _H)), _vec_spec(), _vec_spec(), _vec_spec(),
                  _full((_H, _H)), _vec_spec(), _vec_spec(), _vec_spec()],
        out_specs=[pl.BlockSpec((_BE, _HC), lambda i: (i, 0)),
                   pl.BlockSpec((_BE, _HC), lambda i: (i, 0))],
        out_shape=[jax.ShapeDtypeStruct((_E, _HC), jnp.float32),
                   jax.ShapeDtypeStruct((_E, _HC), jnp.float32)],
        compiler_params=_SEQ,
    )(bases, w1, b1, a1, c1, w2, b2, a2, c2)


_NBLK = 2000
_NG = _N // _NBLK


def _nblk(w):
    return pl.BlockSpec((_NBLK, w), lambda i: (i, 0))


def _node0(h, w0, b0, wp, bp):
    return pl.pallas_call(
        _node0_body,
        grid=(_NG,),
        in_specs=[_nblk(_DIN), _full((_DIN, _H)), _vec_spec(),
                  _full((_H, _H)), _vec_spec()],
        out_specs=[_nblk(_H), _nblk(_HC), _nblk(_HC)],
        out_shape=[jax.ShapeDtypeStruct((_N, _H), jnp.float32),
                   jax.ShapeDtypeStruct((_N, _HC), jnp.float32),
                   jax.ShapeDtypeStruct((_N, _HC), jnp.float32)],
        compiler_params=_SEQ,
    )(h, w0, b0, wp, bp)


def _ffn_a(h, y0, y1, s0, s1, wf1, bf1):
    return pl.pallas_call(
        _ffn_a_body,
        grid=(_NG,),
        in_specs=[_nblk(_H), _nblk(_HC), _nblk(_HC), _nblk(_HC), _nblk(_HC),
                  _full((_H, _H)), _vec_spec()],
        out_specs=[_nblk(_H), _nblk(_H), _full((8, _H))],
        out_shape=[jax.ShapeDtypeStruct((_N, _H), jnp.float32),
                   jax.ShapeDtypeStruct((_N, _H), jnp.float32),
                   jax.ShapeDtypeStruct((8, _H), jnp.float32)],
        compiler_params=_SEQ,
    )(h, y0, y1, s0, s1, wf1, bf1)


def _ffn_b(z1, a, c, wf2, bf2):
    return pl.pallas_call(
        _ffn_b_body,
        grid=(_NG,),
        in_specs=[_nblk(_H), _vec_spec(), _vec_spec(),
                  _full((_H, _H)), _vec_spec()],
        out_specs=[_nblk(_H), _full((8, _H))],
        out_shape=[jax.ShapeDtypeStruct((_N, _H), jnp.float32),
                   jax.ShapeDtypeStruct((8, _H), jnp.float32)],
        compiler_params=_SEQ,
    )(z1, a, c, wf2, bf2)


def _ffn_c(x, z2, a, c, wp, bp):
    return pl.pallas_call(
        _ffn_c_body,
        grid=(_NG,),
        in_specs=[_nblk(_H), _nblk(_H), _vec_spec(), _vec_spec(),
                  _full((_H, _H)), _vec_spec()],
        out_specs=[_nblk(_H), _nblk(_HC), _nblk(_HC)],
        out_shape=[jax.ShapeDtypeStruct((_N, _H), jnp.float32),
                   jax.ShapeDtypeStruct((_N, _HC), jnp.float32),
                   jax.ShapeDtypeStruct((_N, _HC), jnp.float32)],
        compiler_params=_SEQ,
    )(x, z2, a, c, wp, bp)


def _ffn_cl(x, z2, a, c):
    return pl.pallas_call(
        _ffn_cl_body,
        grid=(_NG,),
        in_specs=[_nblk(_H), _nblk(_H), _vec_spec(), _vec_spec()],
        out_specs=_full((8, _H)),
        out_shape=jax.ShapeDtypeStruct((8, _H), jnp.float32),
        compiler_params=_SEQ,
    )(x, z2, a, c)


def _head(nr, w1, b1, w2, b2):
    return pl.pallas_call(
        _head_body,
        out_shape=jax.ShapeDtypeStruct((1, 128), jnp.float32),
    )(nr, w1, b1, w2, b2)


@functools.cache
def _mesh():
    return plsc.VectorSubcoreMesh(core_axis_name="c", subcore_axis_name="s",
                                  num_cores=_NCORE, num_subcores=_NSUB)


def _sc_denom_body(e0, e1, dstv, zeros, s0, s1, idx_v, val_v, acc):
    c = lax.axis_index("c")
    s = lax.axis_index("s")
    r0 = s * _RPT
    pltpu.sync_copy(zeros.at[pl.ds(r0, _RPT)], acc.at[pl.ds(r0, _RPT)])
    plsc.subcore_barrier()
    base = s * _EPT

    def chunk(j, carry):
        off = base + j * _K
        pltpu.sync_copy(dstv.at[pl.ds(off, _K)], idx_v)

        @pl.when(c == 0)
        def _():
            pltpu.sync_copy(e0.at[pl.ds(off, _K)], val_v)

        @pl.when(c == 1)
        def _():
            pltpu.sync_copy(e1.at[pl.ds(off, _K)], val_v)

        pltpu.sync_copy(val_v, acc.at[idx_v], add=True)
        return carry

    lax.fori_loop(0, _EPT // _K, chunk, 0)
    plsc.subcore_barrier()

    @pl.when(c == 0)
    def _():
        pltpu.sync_copy(acc.at[pl.ds(r0, _RPT)], s0.at[pl.ds(r0, _RPT)])

    @pl.when(c == 1)
    def _():
        pltpu.sync_copy(acc.at[pl.ds(r0, _RPT)], s1.at[pl.ds(r0, _RPT)])


@functools.cache
def _sc_denom_kernel():
    return pl.kernel(
        _sc_denom_body,
        out_type=[jax.ShapeDtypeStruct((_NPAD, _HC), jnp.float32),
                  jax.ShapeDtypeStruct((_NPAD, _HC), jnp.float32)],
        mesh=_mesh(),
        scratch_types=[pltpu.VMEM((_K,), jnp.int32),
                       pltpu.VMEM((_K, _HC), jnp.float32),
                       pltpu.VMEM_SHARED((_NPAD, _HC), jnp.float32)],
    )


def _sc_denom(e0, e1, dst, zeros):
    return _sc_denom_kernel()(e0, e1, dst, zeros)


def _sc_agg_body(xp0, xp1, e0, e1, srcv, dstv, zeros, y0, y1,
                 sidx, didx, rows, ev, acc, sem):
    c = lax.axis_index("c")
    s = lax.axis_index("s")
    r0 = s * _RPT
    pltpu.sync_copy(zeros.at[pl.ds(r0, _RPT)], acc.at[pl.ds(r0, _RPT)])
    plsc.subcore_barrier()
    base = s * _EPT

    def chunk(j, carry):
        off = base + j * _K
        pltpu.sync_copy(srcv.at[pl.ds(off, _K)], sidx)
        pltpu.sync_copy(dstv.at[pl.ds(off, _K)], didx)

        @pl.when(c == 0)
        def _():
            pltpu.async_copy(xp0.at[sidx], rows, sem).wait()
            pltpu.sync_copy(e0.at[pl.ds(off, _K)], ev)

        @pl.when(c == 1)
        def _():
            pltpu.async_copy(xp1.at[sidx], rows, sem).wait()
            pltpu.sync_copy(e1.at[pl.ds(off, _K)], ev)

        def rowmul(r, carry2):
            rr = rows.at[r]
            er = ev.at[r]
            for k in range(_HC // 16):
                sl = pl.ds(k * 16, 16)
                rr[sl] = rr[sl] * er[sl]
            return carry2

        lax.fori_loop(0, _K, rowmul, 0)
        pltpu.sync_copy(rows, acc.at[didx], add=True)
        return carry

    lax.fori_loop(0, _EPT // _K, chunk, 0)
    plsc.subcore_barrier()

    @pl.when(c == 0)
    def _():
        pltpu.sync_copy(acc.at[pl.ds(r0, _RPT)], y0.at[pl.ds(r0, _RPT)])

    @pl.when(c == 1)
    def _():
        pltpu.sync_copy(acc.at[pl.ds(r0, _RPT)], y1.at[pl.ds(r0, _RPT)])


@functools.cache
def _sc_agg_kernel():
    return pl.kernel(
        _sc_agg_body,
        out_type=[jax.ShapeDtypeStruct((_NPAD, _HC), jnp.float32),
                  jax.ShapeDtypeStruct((_NPAD, _HC), jnp.float32)],
        mesh=_mesh(),
        scratch_types=[pltpu.VMEM((_K,), jnp.int32),
                       pltpu.VMEM((_K,), jnp.int32),
                       pltpu.VMEM((_K, _HC), jnp.float32),
                       pltpu.VMEM((_K, _HC), jnp.float32),
                       pltpu.VMEM_SHARED((_NPAD, _HC), jnp.float32),
                       pltpu.SemaphoreType.DMA],
    )


def _sc_agg(xp0, xp1, e0, e1, src, dst, zeros):
    return _sc_agg_kernel()(xp0, xp1, e0, e1, src, dst, zeros)


def _bn_coeffs(stats, g, be, cnt):
    mu = stats[0] / cnt
    var = stats[1] / cnt - mu * mu
    a = g * lax.rsqrt(var + 1e-5)
    return a.reshape(1, _H), (be - mu * a).reshape(1, _H)


def kernel(h, edge_index, bases, params):
    p = params
    fe = p["fe"]
    src = edge_index[0]
    dst = edge_index[1]
    zeros = jnp.zeros((_NPAD, _HC), jnp.float32)
    w1 = fe["W1"]
    b1 = fe["b1"].reshape(1, _H)
    w2 = fe["W2"]
    b2 = fe["b2"].reshape(1, _H)

    st1 = _edge_stats1(bases, w1, b1)
    a1, c1 = _bn_coeffs(st1, fe["g1"], fe["be1"], _E)
    st2 = _edge_stats2(bases, w1, b1, a1, c1, w2, b2)
    a2, c2 = _bn_coeffs(st2, fe["g2"], fe["be2"], _E)
    e0, e1 = _edge_e(bases, w1, b1, a1, c1, w2, b2, a2, c2)

    s0, s1 = _sc_denom(e0, e1, dst, zeros)

    convs = p["convs"]
    hcur, xp0, xp1 = _node0(h, p["W0"], p["b0"].reshape(1, _H),
                            convs[0]["Wp"], convs[0]["bp"].reshape(1, _H))
    for l in range(4):
        y0, y1 = _sc_agg(xp0, xp1, e0, e1, src, dst, zeros)
        c = convs[l]
        x, z1, stf1 = _ffn_a(hcur, y0, y1, s0, s1, c["Wf1"],
                             c["bf1"].reshape(1, _H))
        af1, cf1 = _bn_coeffs(stf1, c["gf1"], c["bef1"], _N)
        z2, stf2 = _ffn_b(z1, af1, cf1, c["Wf2"], c["bf2"].reshape(1, _H))
        af2, cf2 = _bn_coeffs(stf2, c["gf2"], c["bef2"], _N)
        if l < 3:
            hcur, xp0, xp1 = _ffn_c(x, z2, af2, cf2, convs[l + 1]["Wp"],
                                    convs[l + 1]["bp"].reshape(1, _H))
        else:
            nr = _ffn_cl(x, z2, af2, cf2)
            out = _head(nr, p["W1"], p["b1"].reshape(1, _H),
                        p["W2"], p["b2"].reshape(1, 128))
    return out


# speedup vs baseline: 5.4060x; 1.8922x over previous
import functools

import jax
import jax.numpy as jnp
from jax import lax
from jax.experimental import pallas as pl
from jax.experimental.pallas import tpu as pltpu
from jax.experimental.pallas import tpu_sc as plsc

_N = 10000
_E = 320000
_DIN = 128
_H = 256
_NB = 16
_HC = 128
_NCORE = 2
_NSUB = 16
_K = 40
_EPT = _E // _NSUB
_NPAD = 10240
_RPT = _NPAD // _NSUB
_BE = 8000


def _gelu(x):
    return 0.5 * x * (1.0 + lax.erf(x * 0.7071067811865476))


def _stats1_body(bases_ref, w1_ref, b1_ref, out_ref):
    i = pl.program_id(0)
    z = jnp.dot(bases_ref[...], w1_ref[...],
                preferred_element_type=jnp.float32) + b1_ref[...]

    @pl.when(i == 0)
    def _():
        out_ref[...] = jnp.zeros_like(out_ref)

    out_ref[0:1, :] += jnp.sum(z, axis=0, keepdims=True)
    out_ref[1:2, :] += jnp.sum(z * z, axis=0, keepdims=True)


def _stats2_body(bases_ref, w1_ref, b1_ref, a1_ref, c1_ref, w2_ref, b2_ref,
                 out_ref):
    i = pl.program_id(0)
    z1 = jnp.dot(bases_ref[...], w1_ref[...],
                 preferred_element_type=jnp.float32) + b1_ref[...]
    y1 = _gelu(z1 * a1_ref[...] + c1_ref[...])
    z2 = jnp.dot(y1, w2_ref[...],
                 preferred_element_type=jnp.float32) + b2_ref[...]

    @pl.when(i == 0)
    def _():
        out_ref[...] = jnp.zeros_like(out_ref)

    out_ref[0:1, :] += jnp.sum(z2, axis=0, keepdims=True)
    out_ref[1:2, :] += jnp.sum(z2 * z2, axis=0, keepdims=True)


def _edge_e_body(bases_ref, w1_ref, b1_ref, a1_ref, c1_ref, w2_ref, b2_ref,
                 a2_ref, c2_ref, e0_ref, e1_ref):
    z1 = jnp.dot(bases_ref[...], w1_ref[...],
                 preferred_element_type=jnp.float32) + b1_ref[...]
    y1 = _gelu(z1 * a1_ref[...] + c1_ref[...])
    z2 = jnp.dot(y1, w2_ref[...],
                 preferred_element_type=jnp.float32) + b2_ref[...]
    e = jnp.exp(_gelu(z2 * a2_ref[...] + c2_ref[...]))
    e0_ref[...] = e[:, :_HC]
    e1_ref[...] = e[:, _HC:]


def _node0_body(h_ref, w0_ref, b0_ref, wp_ref, bp_ref,
                h0_ref, xp0_ref, xp1_ref):
    h0 = jnp.dot(h_ref[...], w0_ref[...],
                 preferred_element_type=jnp.float32) + b0_ref[...]
    h0_ref[...] = h0
    xp = _gelu(jnp.dot(h0, wp_ref[...],
                       preferred_element_type=jnp.float32) + bp_ref[...])
    xp0_ref[...] = xp[:, :_HC]
    xp1_ref[...] = xp[:, _HC:]


def _ffn_a_body(h_ref, y0_ref, y1_ref, s0_ref, s1_ref, wf1_ref, bf1_ref,
                x_ref, z1_ref, st_ref):
    s0 = s0_ref[...]
    s1 = s1_ref[...]
    yn0 = y0_ref[...] * jnp.where(s0 > 0, 1.0 / s0, 0.0)
    yn1 = y1_ref[...] * jnp.where(s1 > 0, 1.0 / s1, 0.0)
    x = h_ref[...] + jnp.concatenate([yn0, yn1], axis=1)
    x_ref[...] = x
    z1 = jnp.dot(x, wf1_ref[...],
                 preferred_element_type=jnp.float32) + bf1_ref[...]
    z1_ref[...] = z1

    @pl.when(pl.program_id(0) == 0)
    def _():
        st_ref[...] = jnp.zeros_like(st_ref)

    st_ref[0:1, :] += jnp.sum(z1, axis=0, keepdims=True)
    st_ref[1:2, :] += jnp.sum(z1 * z1, axis=0, keepdims=True)


def _ffn_b_body(z1_ref, a_ref, c_ref, wf2_ref, bf2_ref, z2_ref, st_ref):
    t = _gelu(z1_ref[...] * a_ref[...] + c_ref[...])
    z2 = jnp.dot(t, wf2_ref[...],
                 preferred_element_type=jnp.float32) + bf2_ref[...]
    z2_ref[...] = z2

    @pl.when(pl.program_id(0) == 0)
    def _():
        st_ref[...] = jnp.zeros_like(st_ref)

    st_ref[0:1, :] += jnp.sum(z2, axis=0, keepdims=True)
    st_ref[1:2, :] += jnp.sum(z2 * z2, axis=0, keepdims=True)


def _ffn_c_body(x_ref, z2_ref, a_ref, c_ref, wp_ref, bp_ref,
                h_ref, xp0_ref, xp1_ref):
    hn = x_ref[...] + _gelu(z2_ref[...] * a_ref[...] + c_ref[...])
    h_ref[...] = hn
    xp = _gelu(jnp.dot(hn, wp_ref[...],
                       preferred_element_type=jnp.float32) + bp_ref[...])
    xp0_ref[...] = xp[:, :_HC]
    xp1_ref[...] = xp[:, _HC:]


def _ffn_cl_body(x_ref, z2_ref, a_ref, c_ref, nr_ref):
    hn = x_ref[...] + _gelu(z2_ref[...] * a_ref[...] + c_ref[...])

    @pl.when(pl.program_id(0) == 0)
    def _():
        nr_ref[...] = jnp.zeros_like(nr_ref)

    nr_ref[0:1, :] += jnp.sum(hn, axis=0, keepdims=True)


def _head_body(nr_ref, w1_ref, b1_ref, w2_ref, b2_ref, out_ref):
    nr = _gelu(jnp.dot(nr_ref[0:1, :], w1_ref[...],
                       preferred_element_type=jnp.float32) + b1_ref[...])
    out_ref[...] = jnp.dot(nr, w2_ref[...],
                           preferred_element_type=jnp.float32) + b2_ref[...]


def _vec_spec(n=_H):
    return pl.BlockSpec((1, n), lambda i: (0, 0))


def _full(shape):
    return pl.BlockSpec(shape, lambda i: tuple(0 for _ in shape))


_SEQ = pltpu.CompilerParams(dimension_semantics=("arbitrary",))


def _edge_stats1(bases, w1, b1):
    return pl.pallas_call(
        _stats1_body,
        grid=(_E // _BE,),
        in_specs=[pl.BlockSpec((_BE, _NB), lambda i: (i, 0)),
                  _full((_NB, _H)), _vec_spec()],
        out_specs=_full((8, _H)),
        out_shape=jax.ShapeDtypeStruct((8, _H), jnp.float32),
        compiler_params=_SEQ,
    )(bases, w1, b1)


def _edge_stats2(bases, w1, b1, a1, c1, w2, b2):
    return pl.pallas_call(
        _stats2_body,
        grid=(_E // _BE,),
        in_specs=[pl.BlockSpec((_BE, _NB), lambda i: (i, 0)),
                  _full((_NB, _H)), _vec_spec(), _vec_spec(), _vec_spec(),
                  _full((_H, _H)), _vec_spec()],
        out_specs=_full((8, _H)),
        out_shape=jax.ShapeDtypeStruct((8, _H), jnp.float32),
        compiler_params=_SEQ,
    )(bases, w1, b1, a1, c1, w2, b2)


def _edge_e(bases, w1, b1, a1, c1, w2, b2, a2, c2):
    return pl.pallas_call(
        _edge_e_body,
        grid=(_E // _BE,),
        in_specs=[pl.BlockSpec((_BE, _NB), lambda i: (i, 0)),
                  _full((_NB, _H)), _vec_spec(), _vec_spec(), _vec_spec(),
                  _full((_H, _H)), _vec_spec(), _vec_spec(), _vec_spec()],
        out_specs=[pl.BlockSpec((_BE, _HC), lambda i: (i, 0)),
                   pl.BlockSpec((_BE, _HC), lambda i: (i, 0))],
        out_shape=[jax.ShapeDtypeStruct((_E, _HC), jnp.float32),
                   jax.ShapeDtypeStruct((_E, _HC), jnp.float32)],
        compiler_params=_SEQ,
    )(bases, w1, b1, a1, c1, w2, b2, a2, c2)


_NBLK = 2000
_NG = _N // _NBLK


def _nblk(w):
    return pl.BlockSpec((_NBLK, w), lambda i: (i, 0))


def _node0(h, w0, b0, wp, bp):
    return pl.pallas_call(
        _node0_body,
        grid=(_NG,),
        in_specs=[_nblk(_DIN), _full((_DIN, _H)), _vec_spec(),
                  _full((_H, _H)), _vec_spec()],
        out_specs=[_nblk(_H), _nblk(_HC), _nblk(_HC)],
        out_shape=[jax.ShapeDtypeStruct((_N, _H), jnp.float32),
                   jax.ShapeDtypeStruct((_N, _HC), jnp.float32),
                   jax.ShapeDtypeStruct((_N, _HC), jnp.float32)],
        compiler_params=_SEQ,
    )(h, w0, b0, wp, bp)


def _ffn_a(h, y0, y1, s0, s1, wf1, bf1):
    return pl.pallas_call(
        _ffn_a_body,
        grid=(_NG,),
        in_specs=[_nblk(_H), _nblk(_HC), _nblk(_HC), _nblk(_HC), _nblk(_HC),
                  _full((_H, _H)), _vec_spec()],
        out_specs=[_nblk(_H), _nblk(_H), _full((8, _H))],
        out_shape=[jax.ShapeDtypeStruct((_N, _H), jnp.float32),
                   jax.ShapeDtypeStruct((_N, _H), jnp.float32),
                   jax.ShapeDtypeStruct((8, _H), jnp.float32)],
        compiler_params=_SEQ,
    )(h, y0, y1, s0, s1, wf1, bf1)


def _ffn_b(z1, a, c, wf2, bf2):
    return pl.pallas_call(
        _ffn_b_body,
        grid=(_NG,),
        in_specs=[_nblk(_H), _vec_spec(), _vec_spec(),
                  _full((_H, _H)), _vec_spec()],
        out_specs=[_nblk(_H), _full((8, _H))],
        out_shape=[jax.ShapeDtypeStruct((_N, _H), jnp.float32),
                   jax.ShapeDtypeStruct((8, _H), jnp.float32)],
        compiler_params=_SEQ,
    )(z1, a, c, wf2, bf2)


def _ffn_c(x, z2, a, c, wp, bp):
    return pl.pallas_call(
        _ffn_c_body,
        grid=(_NG,),
        in_specs=[_nblk(_H), _nblk(_H), _vec_spec(), _vec_spec(),
                  _full((_H, _H)), _vec_spec()],
        out_specs=[_nblk(_H), _nblk(_HC), _nblk(_HC)],
        out_shape=[jax.ShapeDtypeStruct((_N, _H), jnp.float32),
                   jax.ShapeDtypeStruct((_N, _HC), jnp.float32),
                   jax.ShapeDtypeStruct((_N, _HC), jnp.float32)],
        compiler_params=_SEQ,
    )(x, z2, a, c, wp, bp)


def _ffn_cl(x, z2, a, c):
    return pl.pallas_call(
        _ffn_cl_body,
        grid=(_NG,),
        in_specs=[_nblk(_H), _nblk(_H), _vec_spec(), _vec_spec()],
        out_specs=_full((8, _H)),
        out_shape=jax.ShapeDtypeStruct((8, _H), jnp.float32),
        compiler_params=_SEQ,
    )(x, z2, a, c)


def _head(nr, w1, b1, w2, b2):
    return pl.pallas_call(
        _head_body,
        out_shape=jax.ShapeDtypeStruct((1, 128), jnp.float32),
    )(nr, w1, b1, w2, b2)


@functools.cache
def _mesh():
    return plsc.VectorSubcoreMesh(core_axis_name="c", subcore_axis_name="s",
                                  num_cores=_NCORE, num_subcores=_NSUB)


_NCH = _EPT // _K


def _sc_denom_body(e0, e1, dstv, zeros, s0, s1,
                   d0, d1, d2, d3, v0, v1, v2, v3, acc,
                   sd0, sd1, sd2, sd3, se0, se1, se2, se3,
                   ss0, ss1, ss2, ss3):
    c = lax.axis_index("c")
    s = lax.axis_index("s")
    row0 = s * _RPT
    pltpu.sync_copy(zeros.at[pl.ds(row0, _RPT)], acc.at[pl.ds(row0, _RPT)])
    plsc.subcore_barrier()
    base = s * _EPT
    didx = (d0, d1, d2, d3)
    ev = (v0, v1, v2, v3)
    semd = (sd0, sd1, sd2, sd3)
    seme = (se0, se1, se2, se3)
    semsc = (ss0, ss1, ss2, ss3)

    def issue_in(j, b):
        off = base + j * _K
        pltpu.async_copy(dstv.at[pl.ds(off, _K)], didx[b], semd[b])

        @pl.when(c == 0)
        def _():
            pltpu.async_copy(e0.at[pl.ds(off, _K)], ev[b], seme[b])

        @pl.when(c == 1)
        def _():
            pltpu.async_copy(e1.at[pl.ds(off, _K)], ev[b], seme[b])

    def wait_in(b):
        pltpu.make_async_copy(dstv.at[pl.ds(0, _K)], didx[b], semd[b]).wait()
        pltpu.make_async_copy(e0.at[pl.ds(0, _K)], ev[b], seme[b]).wait()

    def wait_scatter(b):
        pltpu.make_async_copy(ev[b], acc.at[didx[b]], semsc[b]).wait()

    issue_in(0, 0)
    issue_in(1, 1)

    def body(it, carry):
        g = it * 4
        for b in range(4):
            j = g + b

            @pl.when(j >= 2)
            def _():
                wait_scatter((b + 2) % 4)

            wait_in(b)
            pltpu.async_copy(ev[b], acc.at[didx[b]], semsc[b], add=True)

            @pl.when(j + 2 < _NCH)
            def _():
                issue_in(j + 2, (b + 2) % 4)

        return carry

    lax.fori_loop(0, _NCH // 4, body, 0)
    wait_scatter(2)
    wait_scatter(3)
    plsc.subcore_barrier()

    @pl.when(c == 0)
    def _():
        pltpu.sync_copy(acc.at[pl.ds(row0, _RPT)], s0.at[pl.ds(row0, _RPT)])

    @pl.when(c == 1)
    def _():
        pltpu.sync_copy(acc.at[pl.ds(row0, _RPT)], s1.at[pl.ds(row0, _RPT)])


@functools.cache
def _sc_denom_kernel():
    return pl.kernel(
        _sc_denom_body,
        out_type=[jax.ShapeDtypeStruct((_NPAD, _HC), jnp.float32),
                  jax.ShapeDtypeStruct((_NPAD, _HC), jnp.float32)],
        mesh=_mesh(),
        scratch_types=[pltpu.VMEM((_K,), jnp.int32)] * 4
                      + [pltpu.VMEM((_K, _HC), jnp.float32)] * 4
                      + [pltpu.VMEM_SHARED((_NPAD, _HC), jnp.float32)]
                      + [pltpu.SemaphoreType.DMA] * 12,
    )


def _sc_denom(e0, e1, dst, zeros):
    return _sc_denom_kernel()(e0, e1, dst, zeros)


def _sc_agg_body(xp0, xp1, e0, e1, srcv, dstv, zeros, y0, y1,
                 t0, t1, t2, t3, d0, d1, d2, d3,
                 r0, r1, r2, r3, v0, v1, v2, v3, acc,
                 si0, si1, si2, si3, sd0, sd1, sd2, sd3,
                 se0, se1, se2, se3, sg0, sg1, sg2, sg3,
                 ss0, ss1, ss2, ss3):
    c = lax.axis_index("c")
    s = lax.axis_index("s")
    row0 = s * _RPT
    pltpu.sync_copy(zeros.at[pl.ds(row0, _RPT)], acc.at[pl.ds(row0, _RPT)])
    plsc.subcore_barrier()
    base = s * _EPT
    sidx = (t0, t1, t2, t3)
    didx = (d0, d1, d2, d3)
    rows = (r0, r1, r2, r3)
    ev = (v0, v1, v2, v3)
    semi = (si0, si1, si2, si3)
    semd = (sd0, sd1, sd2, sd3)
    seme = (se0, se1, se2, se3)
    semg = (sg0, sg1, sg2, sg3)
    semsc = (ss0, ss1, ss2, ss3)

    def issue_in(j, b):
        off = base + j * _K
        pltpu.async_copy(srcv.at[pl.ds(off, _K)], sidx[b], semi[b])
        pltpu.async_copy(dstv.at[pl.ds(off, _K)], didx[b], semd[b])

        @pl.when(c == 0)
        def _():
            pltpu.async_copy(e0.at[pl.ds(off, _K)], ev[b], seme[b])

        @pl.when(c == 1)
        def _():
            pltpu.async_copy(e1.at[pl.ds(off, _K)], ev[b], seme[b])

    def issue_gather(b):
        @pl.when(c == 0)
        def _():
            pltpu.async_copy(xp0.at[sidx[b]], rows[b], semg[b])

        @pl.when(c == 1)
        def _():
            pltpu.async_copy(xp1.at[sidx[b]], rows[b], semg[b])

    def mul(b):
        def rowmul(r, carry2):
            rr = rows[b].at[r]
            er = ev[b].at[r]
            for k in range(_HC // 16):
                sl = pl.ds(k * 16, 16)
                rr[sl] = rr[sl] * er[sl]
            return carry2

        lax.fori_loop(0, _K, rowmul, 0)

    def wait_scatter(b):
        pltpu.make_async_copy(rows[b], acc.at[didx[b]], semsc[b]).wait()

    issue_in(0, 0)
    issue_in(1, 1)

    def body(it, carry):
        g = it * 4
        for p in (0, 1):
            for b in (2 * p, 2 * p + 1):
                j = g + b

                @pl.when(j >= 2)
                def _():
                    wait_scatter((b + 2) % 4)

                pltpu.make_async_copy(srcv.at[pl.ds(0, _K)], sidx[b],
                                      semi[b]).wait()
                issue_gather(b)
            for b in (2 * p, 2 * p + 1):
                j = g + b
                pltpu.make_async_copy(xp0.at[sidx[b]], rows[b],
                                      semg[b]).wait()
                pltpu.make_async_copy(e0.at[pl.ds(0, _K)], ev[b],
                                      seme[b]).wait()
                mul(b)
                pltpu.make_async_copy(dstv.at[pl.ds(0, _K)], didx[b],
                                      semd[b]).wait()
                pltpu.async_copy(rows[b], acc.at[didx[b]], semsc[b],
                                 add=True)

                @pl.when(j + 2 < _NCH)
                def _():
                    issue_in(j + 2, (b + 2) % 4)

        return carry

    lax.fori_loop(0, _NCH // 4, body, 0)
    wait_scatter(2)
    wait_scatter(3)
    plsc.subcore_barrier()

    @pl.when(c == 0)
    def _():
        pltpu.sync_copy(acc.at[pl.ds(row0, _RPT)], y0.at[pl.ds(row0, _RPT)])

    @pl.when(c == 1)
    def _():
        pltpu.sync_copy(acc.at[pl.ds(row0, _RPT)], y1.at[pl.ds(row0, _RPT)])


@functools.cache
def _sc_agg_kernel():
    return pl.kernel(
        _sc_agg_body,
        out_type=[jax.ShapeDtypeStruct((_NPAD, _HC), jnp.float32),
                  jax.ShapeDtypeStruct((_NPAD, _HC), jnp.float32)],
        mesh=_mesh(),
        scratch_types=[pltpu.VMEM((_K,), jnp.int32)] * 8
                      + [pltpu.VMEM((_K, _HC), jnp.float32)] * 8
                      + [pltpu.VMEM_SHARED((_NPAD, _HC), jnp.float32)]
                      + [pltpu.SemaphoreType.DMA] * 20,
    )


def _sc_agg(xp0, xp1, e0, e1, src, dst, zeros):
    return _sc_agg_kernel()(xp0, xp1, e0, e1, src, dst, zeros)


def _bn_coeffs(stats, g, be, cnt):
    mu = stats[0] / cnt
    var = stats[1] / cnt - mu * mu
    a = g * lax.rsqrt(var + 1e-5)
    return a.reshape(1, _H), (be - mu * a).reshape(1, _H)


def kernel(h, edge_index, bases, params):
    p = params
    fe = p["fe"]
    src = edge_index[0]
    dst = edge_index[1]
    zeros = jnp.zeros((_NPAD, _HC), jnp.float32)
    w1 = fe["W1"]
    b1 = fe["b1"].reshape(1, _H)
    w2 = fe["W2"]
    b2 = fe["b2"].reshape(1, _H)

    st1 = _edge_stats1(bases, w1, b1)
    a1, c1 = _bn_coeffs(st1, fe["g1"], fe["be1"], _E)
    st2 = _edge_stats2(bases, w1, b1, a1, c1, w2, b2)
    a2, c2 = _bn_coeffs(st2, fe["g2"], fe["be2"], _E)
    e0, e1 = _edge_e(bases, w1, b1, a1, c1, w2, b2, a2, c2)

    s0, s1 = _sc_denom(e0, e1, dst, zeros)

    convs = p["convs"]
    hcur, xp0, xp1 = _node0(h, p["W0"], p["b0"].reshape(1, _H),
                            convs[0]["Wp"], convs[0]["bp"].reshape(1, _H))
    for l in range(4):
        y0, y1 = _sc_agg(xp0, xp1, e0, e1, src, dst, zeros)
        c = convs[l]
        x, z1, stf1 = _ffn_a(hcur, y0, y1, s0, s1, c["Wf1"],
                             c["bf1"].reshape(1, _H))
        af1, cf1 = _bn_coeffs(stf1, c["gf1"], c["bef1"], _N)
        z2, stf2 = _ffn_b(z1, af1, cf1, c["Wf2"], c["bf2"].reshape(1, _H))
        af2, cf2 = _bn_coeffs(stf2, c["gf2"], c["bef2"], _N)
        if l < 3:
            hcur, xp0, xp1 = _ffn_c(x, z2, af2, cf2, convs[l + 1]["Wp"],
                                    convs[l + 1]["bp"].reshape(1, _H))
        else:
            nr = _ffn_cl(x, z2, af2, cf2)
            out = _head(nr, p["W1"], p["b1"].reshape(1, _H),
                        p["W2"], p["b2"].reshape(1, 128))
    return out
